# baseline (speedup 1.0000x reference)
import functools

import jax
import jax.numpy as jnp
from jax import lax
from jax.experimental import pallas as pl
from jax.experimental.pallas import tpu as pltpu
from jax.experimental.pallas import tpu_sc as plsc

B = 16384
D = 64
H = 128

_info = plsc.get_sparse_core_info()
_NC, _NS = _info.num_cores, _info.num_subcores
_NW = _NC * _NS
_BPW = B // _NW


def _gather_body(uid, rid, cid, vid, utab, rtab, ctab, vtab,
                 out_u, out_r, out_c, out_v, idx_v, rows_v, sem):
    wid = lax.axis_index("s") * _NC + lax.axis_index("c")
    base = wid * _BPW
    for idx_hbm, tab_hbm, out_hbm in (
        (uid, utab, out_u), (rid, rtab, out_r),
        (cid, ctab, out_c), (vid, vtab, out_v)):
        pltpu.sync_copy(idx_hbm.at[pl.ds(base, _BPW)], idx_v)
        pltpu.async_copy(tab_hbm.at[idx_v], rows_v, sem).wait()
        pltpu.sync_copy(rows_v, out_hbm.at[pl.ds(base, _BPW)])


_feat_sds = jax.ShapeDtypeStruct((B, D), jnp.float32)

_gather = pl.kernel(
    _gather_body,
    out_type=(_feat_sds, _feat_sds, _feat_sds, _feat_sds),
    mesh=plsc.VectorSubcoreMesh(core_axis_name="c", subcore_axis_name="s"),
    scratch_types=[
        pltpu.VMEM((_BPW,), jnp.int32),
        pltpu.VMEM((_BPW, D), jnp.float32),
        pltpu.SemaphoreType.DMA,
    ],
    compiler_params=pltpu.CompilerParams(use_tc_tiling_on_sc=False),
)


def _mlp_body(u_ref, r_ref, c_ref, v_ref, w1_ref, b1_ref, w2_ref, b2_ref, o_ref):
    x = jnp.concatenate(
        [u_ref[...], r_ref[...], c_ref[...], v_ref[...]], axis=-1)
    h = jnp.dot(x, w1_ref[...], preferred_element_type=jnp.float32)
    h = jnp.maximum(h + b1_ref[...], 0.0)
    o = jnp.dot(h, w2_ref[...], preferred_element_type=jnp.float32)
    o_ref[...] = o + b2_ref[...]


def _mlp(u, r, c, v, W1, b1, W2, b2):
    blk = 2048
    grid = (B // blk,)
    feat_spec = pl.BlockSpec((blk, D), lambda i: (i, 0))
    return pl.pallas_call(
        _mlp_body,
        grid=grid,
        in_specs=[
            feat_spec, feat_spec, feat_spec, feat_spec,
            pl.BlockSpec((4 * D, H), lambda i: (0, 0)),
            pl.BlockSpec((1, H), lambda i: (0, 0)),
            pl.BlockSpec((H, D), lambda i: (0, 0)),
            pl.BlockSpec((1, D), lambda i: (0, 0)),
        ],
        out_specs=pl.BlockSpec((blk, D), lambda i: (i, 0)),
        out_shape=jax.ShapeDtypeStruct((B, D), jnp.float32),
    )(u, r, c, v, W1, b1.reshape(1, H), W2, b2.reshape(1, D))


def kernel(user_id, region, city, item_id_currentview,
           user_table, region_table, city_table, view_table,
           W1, b1, W2, b2):
    u, r, c, v = _gather(user_id, region, city, item_id_currentview,
                         user_table, region_table, city_table, view_table)
    return _mlp(u, r, c, v, W1, b1, W2, b2)


# speedup vs baseline: 1.5374x; 1.5374x over previous
import jax
import jax.numpy as jnp
from jax import lax
from jax.experimental import pallas as pl
from jax.experimental.pallas import tpu as pltpu
from jax.experimental.pallas import tpu_sc as plsc

B = 16384
D = 64
H = 128

_info = plsc.get_sparse_core_info()
_NC, _NS = _info.num_cores, _info.num_subcores
_NW = _NC * _NS
_BPW = B // _NW


def _gather_one_table(idx_hbm, tab_hbm, out_hbm, base, idx_v, idx_s, buf_v, sem):
    pltpu.sync_copy(idx_hbm.at[pl.ds(base, _BPW)], idx_v)

    def grp_body(g, _):
        vec = idx_v[pl.ds(g * 16, 16)]
        for k in range(16):
            row = vec[k]
            pltpu.async_copy(
                tab_hbm.at[pl.ds(row, 1)],
                buf_v.at[pl.ds(g * 16 + k, 1)], sem)
        return 0

    lax.fori_loop(0, _BPW // 16, grp_body, 0)
    pltpu.make_async_copy(tab_hbm.at[pl.ds(0, _BPW)], buf_v, sem).wait()
    pltpu.sync_copy(buf_v, out_hbm.at[pl.ds(base, _BPW)])


def _gather_body(uid, rid, cid, vid, utab, rtab, ctab, vtab,
                 out_u, out_r, out_c, out_v, idx_v, idx_s, buf_v, sem):
    wid = lax.axis_index("s") * _NC + lax.axis_index("c")
    base = wid * _BPW
    for idx_hbm, tab_hbm, out_hbm in (
        (uid, utab, out_u), (rid, rtab, out_r),
        (cid, ctab, out_c), (vid, vtab, out_v)):
        _gather_one_table(idx_hbm, tab_hbm, out_hbm, base,
                          idx_v, idx_s, buf_v, sem)


_feat_sds = jax.ShapeDtypeStruct((B, D), jnp.float32)

_gather = pl.kernel(
    _gather_body,
    out_type=(_feat_sds, _feat_sds, _feat_sds, _feat_sds),
    mesh=plsc.VectorSubcoreMesh(core_axis_name="c", subcore_axis_name="s"),
    scratch_types=[
        pltpu.VMEM((_BPW,), jnp.int32),
        pltpu.SMEM((_BPW,), jnp.int32),
        pltpu.VMEM((_BPW, D), jnp.float32),
        pltpu.SemaphoreType.DMA,
    ],
)


def _mlp_body(u_ref, r_ref, c_ref, v_ref, w1_ref, b1_ref, w2_ref, b2_ref, o_ref):
    x = jnp.concatenate(
        [u_ref[...], r_ref[...], c_ref[...], v_ref[...]], axis=-1)
    h = jnp.dot(x, w1_ref[...], preferred_element_type=jnp.float32)
    h = jnp.maximum(h + b1_ref[...], 0.0)
    o = jnp.dot(h, w2_ref[...], preferred_element_type=jnp.float32)
    o_ref[...] = o + b2_ref[...]


def _mlp(u, r, c, v, W1, b1, W2, b2):
    blk = 2048
    grid = (B // blk,)
    feat_spec = pl.BlockSpec((blk, D), lambda i: (i, 0))
    return pl.pallas_call(
        _mlp_body,
        grid=grid,
        in_specs=[
            feat_spec, feat_spec, feat_spec, feat_spec,
            pl.BlockSpec((4 * D, H), lambda i: (0, 0)),
            pl.BlockSpec((1, H), lambda i: (0, 0)),
            pl.BlockSpec((H, D), lambda i: (0, 0)),
            pl.BlockSpec((1, D), lambda i: (0, 0)),
        ],
        out_specs=pl.BlockSpec((blk, D), lambda i: (i, 0)),
        out_shape=jax.ShapeDtypeStruct((B, D), jnp.float32),
    )(u, r, c, v, W1, b1.reshape(1, H), W2, b2.reshape(1, D))


def kernel(user_id, region, city, item_id_currentview,
           user_table, region_table, city_table, view_table,
           W1, b1, W2, b2):
    u, r, c, v = _gather(user_id, region, city, item_id_currentview,
                         user_table, region_table, city_table, view_table)
    return _mlp(u, r, c, v, W1, b1, W2, b2)
